# baseline (speedup 1.0000x reference)
import functools

import jax
import jax.numpy as jnp
from jax import lax
from jax.experimental import pallas as pl
from jax.experimental.pallas import tpu as pltpu

DELTA_V = 0.2
DELTA_D = 0.2
ALPHA = 1.0
BETA = 1.0
GAMMA = 0.001

NV = 8


def _phase0(feat, gt, t, seg_acc):
    T = feat.shape[1]
    iota_v = lax.broadcasted_iota(jnp.int32, (NV, T), 0)
    onehot = (gt == iota_v).astype(jnp.float32)
    ones = jnp.ones((1, T), jnp.float32)
    featx = jnp.concatenate([feat, ones], axis=0)
    seg = lax.dot_general(
        featx, onehot, (((1,), (1,)), ((), ())),
        preferred_element_type=jnp.float32,
        precision=lax.Precision.DEFAULT)

    @pl.when(t == 0)
    def _():
        seg_acc[...] = seg

    @pl.when(t != 0)
    def _():
        seg_acc[...] += seg


def _phase1(feat, gt, t, c_dim, inv_hw, seg_acc, s_acc):
    T = feat.shape[1]
    cmv = seg_acc[:c_dim, :] * inv_hw
    iota_v = lax.broadcasted_iota(jnp.int32, (NV, T), 0)
    onehot = (gt == iota_v).astype(jnp.float32)
    cmsel = lax.dot_general(
        cmv, onehot, (((1,), (0,)), ((), ())),
        preferred_element_type=jnp.float32,
        precision=lax.Precision.DEFAULT)
    diff = feat - cmsel
    d2sel = jnp.sum(diff * diff, axis=0, keepdims=True)
    norm = jnp.sqrt(d2sel)
    contrib = onehot * norm
    part = jnp.sum(contrib, axis=1, keepdims=True)
    part = jnp.broadcast_to(part, (NV, 128))

    @pl.when(t == 0)
    def _():
        s_acc[...] = part

    @pl.when(t != 0)
    def _():
        s_acc[...] += part


def _assembly(n, c_dim, hw, seg_acc, s_acc, accv, accd, accg, loss_ref,
              cms_ref, n_batch):
    f32 = jnp.float32
    inv_hw = f32(1.0 / hw)
    eye = (lax.broadcasted_iota(jnp.int32, (NV, NV), 0)
           == lax.broadcasted_iota(jnp.int32, (NV, NV), 1)).astype(f32)
    lt = (lax.broadcasted_iota(jnp.int32, (NV, NV), 0)
          < lax.broadcasted_iota(jnp.int32, (NV, NV), 1)).astype(f32)
    i_col = lax.broadcasted_iota(jnp.int32, (NV, NV), 0).astype(f32)
    e_cnt = (lax.broadcasted_iota(jnp.int32, (c_dim + 1, 1), 0)
             == c_dim).astype(f32)
    dot = functools.partial(
        lax.dot_general, preferred_element_type=f32,
        precision=lax.Precision.HIGHEST)

    a_n = seg_acc[...]
    cmv = a_n[:c_dim, :] * inv_hw
    counts_row = a_n[c_dim:c_dim + 1, :]
    pres = (counts_row > 0.0).astype(f32)
    rank = dot(pres, lt, (((1,), (0,)), ((), ())))
    tot = jnp.sum(pres, keepdims=True)
    cond = jnp.logical_or(
        rank == i_col,
        jnp.logical_and(rank == 0.0, i_col >= tot))
    perm = pres * cond.astype(f32)
    cm = dot(cmv, perm, (((1,), (1,)), ((), ())))
    counts_col = dot(a_n, e_cnt, (((0,), (0,)), ((), ())))
    cnt_slot = dot(perm, counts_col, (((1,), (0,)), ((), ())))
    s_col = s_acc[:, 0:1]
    s_slot = dot(perm, s_col, (((1,), (0,)), ((), ())))
    gram = dot(cm, cm, (((0,), (0,)), ((), ())))
    diag_c = jnp.sum(gram * eye, axis=1, keepdims=True)
    diag_r = jnp.sum(gram * eye, axis=0, keepdims=True)
    cmnorm = jnp.sqrt(jnp.maximum(diag_c, 0.0))
    mean_norm = (s_slot + (f32(hw) - cnt_slot) * cmnorm) * inv_hw
    v_n = jnp.sum(jnp.maximum(mean_norm - DELTA_V, 0.0),
                  keepdims=True) * (1.0 / NV)
    dist = jnp.sqrt(jnp.maximum(diag_c + diag_r - 2.0 * gram, 0.0))
    row_mean = jnp.sum(dist, axis=1, keepdims=True) * (1.0 / (NV - 1))
    d_n = jnp.sum(jnp.maximum(2.0 * DELTA_D - row_mean, 0.0), keepdims=True)
    g_n = jnp.sum(cmnorm, keepdims=True) * (1.0 / NV)
    cms_ref[0] = cm

    v_b = jnp.broadcast_to(v_n, (1, 128))
    d_b = jnp.broadcast_to(d_n, (1, 128))
    g_b = jnp.broadcast_to(g_n, (1, 128))

    @pl.when(n == 0)
    def _():
        accv[...] = v_b
        accd[...] = d_b
        accg[...] = g_b

    @pl.when(n != 0)
    def _():
        accv[...] += v_b
        accd[...] += d_b
        accg[...] += g_b

    @pl.when(n == n_batch - 1)
    def _():
        variance = accv[0:1, 0:1] * (1.0 / n_batch)
        distance = accd[0:1, 0:1] * (1.0 / (n_batch * NV))
        normal = accg[0:1, 0:1] * (1.0 / n_batch)
        total = ALPHA * variance + BETA * distance + GAMMA * normal
        row = lax.broadcasted_iota(jnp.int32, (4, 128), 0)
        col = lax.broadcasted_iota(jnp.int32, (4, 128), 1)
        sel = lambda i: jnp.logical_and(row == i, col == 0).astype(f32)
        loss_ref[...] = (total * sel(0) + variance * sel(1)
                         + distance * sel(2) + normal * sel(3))


def _fused_body(feat_ref, gt_ref, loss_ref, cms_ref,
                seg_acc, s_acc, accv, accd, accg, *,
                n_batch, c_dim, hw, nt):
    n = pl.program_id(0)
    ph = pl.program_id(1)
    t = pl.program_id(2)
    feat = feat_ref[0]
    gt = gt_ref[0]

    @pl.when(ph == 0)
    def _():
        _phase0(feat, gt, t, seg_acc)

    @pl.when(ph == 1)
    def _():
        _phase1(feat, gt, t, c_dim, 1.0 / hw, seg_acc, s_acc)

        @pl.when(t == nt - 1)
        def _():
            _assembly(n, c_dim, hw, seg_acc, s_acc, accv, accd, accg,
                      loss_ref, cms_ref, n_batch)


def kernel(features, ground_truth):
    N, C, H, W = features.shape
    P = H * W
    feat = features.reshape(N, C, P)
    gt = ground_truth.reshape(N, 1, P).astype(jnp.int32)

    T = min(P, 131072)
    nt = P // T

    losses, cms = pl.pallas_call(
        functools.partial(_fused_body, n_batch=N, c_dim=C, hw=P, nt=nt),
        grid=(N, 2, nt),
        in_specs=[
            pl.BlockSpec((1, C, T), lambda n, ph, t: (n, 0, t)),
            pl.BlockSpec((1, 1, T), lambda n, ph, t: (n, 0, t)),
        ],
        out_specs=(
            pl.BlockSpec((4, 128), lambda n, ph, t: (0, 0)),
            pl.BlockSpec((1, C, NV), lambda n, ph, t: (n, 0, 0)),
        ),
        out_shape=(
            jax.ShapeDtypeStruct((4, 128), jnp.float32),
            jax.ShapeDtypeStruct((N, C, NV), jnp.float32),
        ),
        scratch_shapes=[
            pltpu.VMEM((C + 1, NV), jnp.float32),
            pltpu.VMEM((NV, 128), jnp.float32),
            pltpu.VMEM((1, 128), jnp.float32),
            pltpu.VMEM((1, 128), jnp.float32),
            pltpu.VMEM((1, 128), jnp.float32),
        ],
    )(feat, gt)

    out = (losses[0, 0], losses[1, 0], losses[2, 0], losses[3, 0])
    return out + tuple(cms[n] for n in range(N))


# speedup vs baseline: 1.0175x; 1.0175x over previous
import functools

import jax
import jax.numpy as jnp
from jax import lax
from jax.experimental import pallas as pl
from jax.experimental.pallas import tpu as pltpu

DELTA_V = 0.2
DELTA_D = 0.2
ALPHA = 1.0
BETA = 1.0
GAMMA = 0.001

NV = 8


def _phase0(feat, gt, t, seg_acc):
    T = feat.shape[1]
    iota_v = lax.broadcasted_iota(jnp.int32, (NV, T), 0)
    onehot = (gt == iota_v).astype(jnp.float32)
    ones = jnp.ones((1, T), jnp.float32)
    featx = jnp.concatenate([feat, ones], axis=0)
    seg = lax.dot_general(
        featx, onehot, (((1,), (1,)), ((), ())),
        preferred_element_type=jnp.float32,
        precision=lax.Precision.DEFAULT)

    @pl.when(t == 0)
    def _():
        seg_acc[...] = seg

    @pl.when(t != 0)
    def _():
        seg_acc[...] += seg


def _phase1(feat, gt, t, c_dim, inv_hw, seg_acc, s_acc):
    T = feat.shape[1]
    cmv = seg_acc[:c_dim, :] * inv_hw
    iota_v = lax.broadcasted_iota(jnp.int32, (NV, T), 0)
    onehot = (gt == iota_v).astype(jnp.float32)
    cmsel = lax.dot_general(
        cmv, onehot, (((1,), (0,)), ((), ())),
        preferred_element_type=jnp.float32,
        precision=lax.Precision.DEFAULT)
    diff = feat - cmsel
    d2sel = jnp.sum(diff * diff, axis=0, keepdims=True)
    norm = jnp.sqrt(d2sel)
    contrib = onehot * norm
    part = jnp.sum(contrib, axis=1, keepdims=True)
    part = jnp.broadcast_to(part, (NV, 128))

    @pl.when(t == 0)
    def _():
        s_acc[...] = part

    @pl.when(t != 0)
    def _():
        s_acc[...] += part


def _assembly(n, c_dim, hw, seg_acc, s_acc, accv, accd, accg, loss_ref,
              cms_ref, n_batch):
    f32 = jnp.float32
    inv_hw = f32(1.0 / hw)
    eye = (lax.broadcasted_iota(jnp.int32, (NV, NV), 0)
           == lax.broadcasted_iota(jnp.int32, (NV, NV), 1)).astype(f32)
    lt = (lax.broadcasted_iota(jnp.int32, (NV, NV), 0)
          < lax.broadcasted_iota(jnp.int32, (NV, NV), 1)).astype(f32)
    i_col = lax.broadcasted_iota(jnp.int32, (NV, NV), 0).astype(f32)
    e_cnt = (lax.broadcasted_iota(jnp.int32, (c_dim + 1, 1), 0)
             == c_dim).astype(f32)
    dot = functools.partial(
        lax.dot_general, preferred_element_type=f32,
        precision=lax.Precision.HIGHEST)

    a_n = seg_acc[...]
    cmv = a_n[:c_dim, :] * inv_hw
    counts_row = a_n[c_dim:c_dim + 1, :]
    pres = (counts_row > 0.0).astype(f32)
    rank = dot(pres, lt, (((1,), (0,)), ((), ())))
    tot = jnp.sum(pres, keepdims=True)
    cond = jnp.logical_or(
        rank == i_col,
        jnp.logical_and(rank == 0.0, i_col >= tot))
    perm = pres * cond.astype(f32)
    cm = dot(cmv, perm, (((1,), (1,)), ((), ())))
    counts_col = dot(a_n, e_cnt, (((0,), (0,)), ((), ())))
    cnt_slot = dot(perm, counts_col, (((1,), (0,)), ((), ())))
    s_col = s_acc[:, 0:1]
    s_slot = dot(perm, s_col, (((1,), (0,)), ((), ())))
    gram = dot(cm, cm, (((0,), (0,)), ((), ())))
    diag_c = jnp.sum(gram * eye, axis=1, keepdims=True)
    diag_r = jnp.sum(gram * eye, axis=0, keepdims=True)
    cmnorm = jnp.sqrt(jnp.maximum(diag_c, 0.0))
    mean_norm = (s_slot + (f32(hw) - cnt_slot) * cmnorm) * inv_hw
    v_n = jnp.sum(jnp.maximum(mean_norm - DELTA_V, 0.0),
                  keepdims=True) * (1.0 / NV)
    dist = jnp.sqrt(jnp.maximum(diag_c + diag_r - 2.0 * gram, 0.0))
    row_mean = jnp.sum(dist, axis=1, keepdims=True) * (1.0 / (NV - 1))
    d_n = jnp.sum(jnp.maximum(2.0 * DELTA_D - row_mean, 0.0), keepdims=True)
    g_n = jnp.sum(cmnorm, keepdims=True) * (1.0 / NV)
    cms_ref[0] = cm

    v_b = jnp.broadcast_to(v_n, (1, 128))
    d_b = jnp.broadcast_to(d_n, (1, 128))
    g_b = jnp.broadcast_to(g_n, (1, 128))

    @pl.when(n == 0)
    def _():
        accv[...] = v_b
        accd[...] = d_b
        accg[...] = g_b

    @pl.when(n != 0)
    def _():
        accv[...] += v_b
        accd[...] += d_b
        accg[...] += g_b

    @pl.when(n == n_batch - 1)
    def _():
        variance = accv[0:1, 0:1] * (1.0 / n_batch)
        distance = accd[0:1, 0:1] * (1.0 / (n_batch * NV))
        normal = accg[0:1, 0:1] * (1.0 / n_batch)
        total = ALPHA * variance + BETA * distance + GAMMA * normal
        row = lax.broadcasted_iota(jnp.int32, (4, 128), 0)
        col = lax.broadcasted_iota(jnp.int32, (4, 128), 1)
        sel = lambda i: jnp.logical_and(row == i, col == 0).astype(f32)
        loss_ref[...] = (total * sel(0) + variance * sel(1)
                         + distance * sel(2) + normal * sel(3))


def _p1_call_body(feat_ref, gt_ref, seg_ref):
    t = pl.program_id(1)
    feat = feat_ref[0]
    gt = gt_ref[0]
    T = feat.shape[1]
    iota_v = lax.broadcasted_iota(jnp.int32, (NV, T), 0)
    onehot = (gt == iota_v).astype(jnp.float32)
    ones = jnp.ones((1, T), jnp.float32)
    featx = jnp.concatenate([feat, ones], axis=0)
    seg = lax.dot_general(
        featx, onehot, (((1,), (1,)), ((), ())),
        preferred_element_type=jnp.float32,
        precision=lax.Precision.DEFAULT)

    @pl.when(t == 0)
    def _():
        seg_ref[0] = seg

    @pl.when(t != 0)
    def _():
        seg_ref[0] += seg


def _p2_call_body(seg_ref, feat_ref, gt_ref, loss_ref, cms_ref,
                  seg_acc, s_acc, accv, accd, accg, *,
                  n_batch, c_dim, hw, nt):
    n = pl.program_id(0)
    t = pl.program_id(1)
    feat = feat_ref[0]
    gt = gt_ref[0]
    seg_acc[...] = seg_ref[0]
    _phase1(feat, gt, t, c_dim, 1.0 / hw, seg_acc, s_acc)

    @pl.when(t == nt - 1)
    def _():
        _assembly(n, c_dim, hw, seg_acc, s_acc, accv, accd, accg,
                  loss_ref, cms_ref, n_batch)


def kernel(features, ground_truth):
    N, C, H, W = features.shape
    P = H * W
    feat = features.reshape(N, C, P)
    gt = ground_truth.reshape(N, 1, P).astype(jnp.int32)

    T = min(P, 131072)
    nt = P // T

    seg = pl.pallas_call(
        _p1_call_body,
        grid=(N, nt),
        in_specs=[
            pl.BlockSpec((1, C, T), lambda n, t: (n, 0, t)),
            pl.BlockSpec((1, 1, T), lambda n, t: (n, 0, t)),
        ],
        out_specs=pl.BlockSpec((1, C + 1, NV), lambda n, t: (n, 0, 0)),
        out_shape=jax.ShapeDtypeStruct((N, C + 1, NV), jnp.float32),
    )(feat, gt)

    losses, cms = pl.pallas_call(
        functools.partial(_p2_call_body, n_batch=N, c_dim=C, hw=P, nt=nt),
        grid=(N, nt),
        in_specs=[
            pl.BlockSpec((1, C + 1, NV), lambda n, t: (n, 0, 0)),
            pl.BlockSpec((1, C, T), lambda n, t: (n, 0, t)),
            pl.BlockSpec((1, 1, T), lambda n, t: (n, 0, t)),
        ],
        out_specs=(
            pl.BlockSpec((4, 128), lambda n, t: (0, 0)),
            pl.BlockSpec((1, C, NV), lambda n, t: (n, 0, 0)),
        ),
        out_shape=(
            jax.ShapeDtypeStruct((4, 128), jnp.float32),
            jax.ShapeDtypeStruct((N, C, NV), jnp.float32),
        ),
        scratch_shapes=[
            pltpu.VMEM((C + 1, NV), jnp.float32),
            pltpu.VMEM((NV, 128), jnp.float32),
            pltpu.VMEM((1, 128), jnp.float32),
            pltpu.VMEM((1, 128), jnp.float32),
            pltpu.VMEM((1, 128), jnp.float32),
        ],
    )(seg, feat, gt)

    out = (losses[0, 0], losses[1, 0], losses[2, 0], losses[3, 0])
    return out + tuple(cms[n] for n in range(N))
